# baseline (speedup 1.0000x reference)
import jax
import jax.numpy as jnp
from jax.experimental import pallas as pl

_B, _L, _CL, _D, _E, _G = 32, 64, 512, 1024, 8, 2
_F = 3 * _CL
_R = _B * _L
_NE = _E + _G
_EPS = 1e-9

_DB = 512
_ND = _D // _DB


def _moe_kernel(logits_ref, mask_ref, flat_ref, w_ref, b_ref, out_ref):
    e = pl.program_id(1)

    logits = logits_ref[...]
    maskf = jnp.where(mask_ref[...] == 1, 1.0, 0.0).astype(jnp.float32)
    g = jax.nn.softmax(logits, axis=1) * maskf
    g = g / (jnp.sum(g, axis=1, keepdims=True) + _EPS)

    grow = jnp.repeat(g, _L, axis=0)
    lane = jax.lax.broadcasted_iota(jnp.int32, (_R, _E), 1)
    scale = jnp.sum(jnp.where(lane == e, grow, 0.0), axis=1, keepdims=True)
    scale = jnp.where(e < _E, scale, 1.0)

    brow_i = jax.lax.broadcasted_iota(jnp.int32, (16, _DB), 0)
    bias = jnp.sum(jnp.where(brow_i == e, b_ref[...], 0.0), axis=0,
                   keepdims=True)

    y = jnp.dot(flat_ref[...], w_ref[0], preferred_element_type=jnp.float32)
    contrib = scale * (y + bias)

    @pl.when(e == 0)
    def _init():
        out_ref[...] = contrib

    @pl.when(e == _E)
    def _general_start():
        rounded = out_ref[...].astype(jnp.bfloat16).astype(jnp.float32)
        out_ref[...] = rounded + contrib

    @pl.when((e != 0) & (e != _E))
    def _accum():
        out_ref[...] += contrib


def kernel(cycle_curve_data, logits, moe_masks, We, be, Wg, bg):
    flat = cycle_curve_data.reshape(_R, _F).astype(jnp.bfloat16)
    w_all = jnp.concatenate([We, Wg], axis=0).astype(jnp.bfloat16)
    b_all = jnp.zeros((16, _D), jnp.float32)
    b_all = b_all.at[:_E].set(be).at[_E:_NE].set(bg)

    out = pl.pallas_call(
        _moe_kernel,
        grid=(_ND, _NE),
        in_specs=[
            pl.BlockSpec((_B, _E), lambda d, e: (0, 0)),
            pl.BlockSpec((_B, _E), lambda d, e: (0, 0)),
            pl.BlockSpec((_R, _F), lambda d, e: (0, 0)),
            pl.BlockSpec((1, _F, _DB), lambda d, e: (e, 0, d)),
            pl.BlockSpec((16, _DB), lambda d, e: (0, d)),
        ],
        out_specs=pl.BlockSpec((_R, _DB), lambda d, e: (0, d)),
        out_shape=jax.ShapeDtypeStruct((_R, _D), jnp.float32),
    )(logits, moe_masks.astype(jnp.int32), flat, w_all, b_all)

    final_out = out.reshape(_B, _L, _D)
    aug_loss = jnp.zeros((), dtype=jnp.float32)
    guide_loss = jnp.zeros((), dtype=jnp.float32)
    return (final_out, aug_loss, guide_loss)


# speedup vs baseline: 1.2879x; 1.2879x over previous
import jax
import jax.numpy as jnp
from jax.experimental import pallas as pl
from jax.experimental.pallas import tpu as pltpu

_B, _L, _CL, _D, _E, _G = 32, 64, 512, 1024, 8, 2
_F = 3 * _CL
_R = _B * _L
_NE = _E + _G
_EPS = 1e-9

_DB = 512
_ND = _D // _DB


def _moe_kernel(logits_ref, mask_ref, flat_ref, we_ref, wg_ref, b_ref,
                out_ref, fbf_ref):
    d = pl.program_id(0)
    e = pl.program_id(1)

    @pl.when((d == 0) & (e == 0))
    def _cast_flat():
        fbf_ref[...] = flat_ref[...].astype(jnp.bfloat16)

    logits = logits_ref[...]
    maskf = jnp.where(mask_ref[...] == 1, 1.0, 0.0).astype(jnp.float32)
    g = jax.nn.softmax(logits, axis=1) * maskf
    g = g / (jnp.sum(g, axis=1, keepdims=True) + _EPS)

    grow = jnp.repeat(g, _L, axis=0)
    lane = jax.lax.broadcasted_iota(jnp.int32, (_R, _E), 1)
    scale = jnp.sum(jnp.where(lane == e, grow, 0.0), axis=1, keepdims=True)
    scale = jnp.where(e < _E, scale, 1.0)

    brow_i = jax.lax.broadcasted_iota(jnp.int32, (16, _DB), 0)
    bias = jnp.sum(jnp.where(brow_i == e, b_ref[...], 0.0), axis=0,
                   keepdims=True)

    w = jnp.where(e < _E, we_ref[0], wg_ref[0]).astype(jnp.bfloat16)
    y = jnp.dot(fbf_ref[...], w, preferred_element_type=jnp.float32)
    contrib = scale * (y + bias)

    @pl.when(e == 0)
    def _init():
        out_ref[...] = contrib

    @pl.when(e == _E)
    def _general_start():
        rounded = out_ref[...].astype(jnp.bfloat16).astype(jnp.float32)
        out_ref[...] = rounded + contrib

    @pl.when((e != 0) & (e != _E))
    def _accum():
        out_ref[...] += contrib


def kernel(cycle_curve_data, logits, moe_masks, We, be, Wg, bg):
    flat = cycle_curve_data.reshape(_R, _F)
    b_all = jnp.zeros((16, _D), jnp.float32)
    b_all = b_all.at[:_E].set(be).at[_E:_NE].set(bg)

    out = pl.pallas_call(
        _moe_kernel,
        grid=(_ND, _NE),
        in_specs=[
            pl.BlockSpec((_B, _E), lambda d, e: (0, 0)),
            pl.BlockSpec((_B, _E), lambda d, e: (0, 0)),
            pl.BlockSpec((_R, _F), lambda d, e: (0, 0)),
            pl.BlockSpec((1, _F, _DB),
                         lambda d, e: (jnp.minimum(e, _E - 1), 0, d)),
            pl.BlockSpec((1, _F, _DB),
                         lambda d, e: (jnp.where(e < _E, 0, e - _E), 0, d)),
            pl.BlockSpec((16, _DB), lambda d, e: (0, d)),
        ],
        out_specs=pl.BlockSpec((_R, _DB), lambda d, e: (0, d)),
        out_shape=jax.ShapeDtypeStruct((_R, _D), jnp.float32),
        scratch_shapes=[pltpu.VMEM((_R, _F), jnp.bfloat16)],
    )(logits, moe_masks.astype(jnp.int32), flat, We, Wg, b_all)

    final_out = out.reshape(_B, _L, _D)
    aug_loss = jnp.zeros((), dtype=jnp.float32)
    guide_loss = jnp.zeros((), dtype=jnp.float32)
    return (final_out, aug_loss, guide_loss)
